# Initial kernel scaffold; baseline (speedup 1.0000x reference)
import jax
import jax.numpy as jnp
from jax.experimental import pallas as pl


def kernel(x, edge_index, W1, b1, gamma, beta, W2, b2):
    raise NotImplementedError("write your pallas kernel here")


# speedup vs baseline: 27.2610x; 27.2610x over previous
import functools

import jax
import jax.numpy as jnp
from jax import lax
from jax.experimental import pallas as pl
from jax.experimental.pallas import tpu as pltpu
from jax.experimental.pallas import tpu_sc as plsc

N = 10000
E = 320000
D = 128
H = 16
C = 10

NP = 10112
CH = 128
NT = 32
CPT = 80
EP = NT * CPT * CH
RP = NP // 16

_mesh = plsc.VectorSubcoreMesh(core_axis_name="c", subcore_axis_name="s")


@functools.partial(
    pl.kernel,
    out_type=(jax.ShapeDtypeStruct((NP,), jnp.float32),
              jax.ShapeDtypeStruct((NP,), jnp.float32)),
    mesh=_mesh,
    scratch_types=[
        pltpu.VMEM((CPT, CH), jnp.int32),
        pltpu.VMEM((CH,), jnp.float32),
        pltpu.VMEM((RP,), jnp.float32),
        pltpu.VMEM_SHARED((NP,), jnp.float32),
    ],
)
def _sc_deg(dst_hbm, zero1_hbm, out0_hbm, out1_hbm, idx_d, ones_v, stripe, acc):
    c = lax.axis_index("c")
    s = lax.axis_index("s")
    wid = c * 16 + s
    pltpu.sync_copy(zero1_hbm.at[pl.ds(s * RP, RP)], stripe)
    pltpu.sync_copy(stripe, acc.at[pl.ds(s * RP, RP)])
    for i in range(CH // 16):
        ones_v[pl.ds(i * 16, 16)] = jnp.full((16,), 1.0, jnp.float32)
    pltpu.sync_copy(dst_hbm.at[pl.ds(wid * CPT, CPT)], idx_d)
    plsc.subcore_barrier()

    def body(j, carry):
        pltpu.sync_copy(ones_v, acc.at[idx_d.at[j]], add=True)
        return carry

    lax.fori_loop(0, CPT, body, 0)
    plsc.subcore_barrier()

    pltpu.sync_copy(acc.at[pl.ds(s * RP, RP)], stripe)

    @pl.when(c == 0)
    def _():
        pltpu.sync_copy(stripe, out0_hbm.at[pl.ds(s * RP, RP)])

    @pl.when(c == 1)
    def _():
        pltpu.sync_copy(stripe, out1_hbm.at[pl.ds(s * RP, RP)])


@functools.partial(
    pl.kernel,
    out_type=jax.ShapeDtypeStruct((2, NP, H), jnp.float32),
    mesh=_mesh,
    scratch_types=[
        pltpu.VMEM((CPT, CH), jnp.int32),
        pltpu.VMEM((CPT, CH), jnp.int32),
        pltpu.VMEM((CH, H), jnp.float32),
        pltpu.VMEM((RP, H), jnp.float32),
        pltpu.VMEM_SHARED((NP, H), jnp.float32),
        pltpu.SemaphoreType.DMA,
    ],
    compiler_params=pltpu.CompilerParams(use_tc_tiling_on_sc=False),
)
def _sc_agg(u_hbm, src_hbm, dst_hbm, zero2_hbm, out_hbm, idx_s, idx_d, rows,
            stripe, acc, sem):
    c = lax.axis_index("c")
    s = lax.axis_index("s")
    wid = c * 16 + s
    pltpu.sync_copy(zero2_hbm.at[pl.ds(s * RP, RP)], stripe)
    pltpu.sync_copy(stripe, acc.at[pl.ds(s * RP, RP)])
    pltpu.sync_copy(src_hbm.at[pl.ds(wid * CPT, CPT)], idx_s)
    pltpu.sync_copy(dst_hbm.at[pl.ds(wid * CPT, CPT)], idx_d)
    plsc.subcore_barrier()

    def body(j, carry):
        pltpu.async_copy(u_hbm.at[idx_s.at[j]], rows, sem).wait()
        pltpu.sync_copy(rows, acc.at[idx_d.at[j]], add=True)
        return carry

    lax.fori_loop(0, CPT, body, 0)
    plsc.subcore_barrier()
    pltpu.sync_copy(acc.at[pl.ds(s * RP, RP)], stripe)
    pltpu.sync_copy(stripe, out_hbm.at[c, pl.ds(s * RP, RP)])


def _tc_a_body(x_ref, w1_ref, p0_ref, p1_ref, u_ref):
    xw = jnp.dot(x_ref[...], w1_ref[...], preferred_element_type=jnp.float32)
    dinv = lax.rsqrt(p0_ref[...] + p1_ref[...] + 1.0)
    u_ref[...] = xw * dinv


def _tc_b_body(a0_ref, a1_ref, u_ref, p0_ref, p1_ref, g_ref, bt_ref, b1_ref,
               w2_ref, v_ref):
    dinv = lax.rsqrt(p0_ref[...] + p1_ref[...] + 1.0)
    h = dinv * (a0_ref[...] + a1_ref[...] + u_ref[...]) + b1_ref[...]
    rowmask = lax.broadcasted_iota(jnp.int32, (NP, H), 0) < N
    hm = jnp.where(rowmask, h, 0.0)
    mean = jnp.sum(hm, axis=0, keepdims=True) * (1.0 / N)
    var = jnp.sum(hm * hm, axis=0, keepdims=True) * (1.0 / N) - mean * mean
    hn = g_ref[...] * (h - mean) * lax.rsqrt(var + 1e-5) + bt_ref[...]
    hr = jnp.maximum(hn, 0.0)
    hw2 = jnp.dot(hr, w2_ref[...], preferred_element_type=jnp.float32)
    v_ref[...] = jnp.where(rowmask, hw2 * dinv, 0.0)


def _tc_c_body(a0_ref, a1_ref, v_ref, p0_ref, p1_ref, b2_ref, out_ref):
    dinv = lax.rsqrt(p0_ref[...] + p1_ref[...] + 1.0)
    o = dinv * (a0_ref[...] + a1_ref[...] + v_ref[...]) + b2_ref[...]
    colmask = lax.broadcasted_iota(jnp.int32, (NP, H), 1) < C
    ol = jnp.where(colmask, o, -1e30)
    m = jnp.max(ol, axis=1, keepdims=True)
    lse = jnp.log(jnp.sum(jnp.exp(ol - m), axis=1, keepdims=True)) + m
    out_ref[...] = ol - lse


_tc_a = pl.pallas_call(
    _tc_a_body, out_shape=jax.ShapeDtypeStruct((NP, H), jnp.float32))
_tc_b = pl.pallas_call(
    _tc_b_body, out_shape=jax.ShapeDtypeStruct((NP, H), jnp.float32))
_tc_c = pl.pallas_call(
    _tc_c_body, out_shape=jax.ShapeDtypeStruct((NP, H), jnp.float32))


def kernel(x, edge_index, W1, b1, gamma, beta, W2, b2):
    src = edge_index[0]
    dst = edge_index[1]
    pad = jnp.full((EP - E,), N, jnp.int32)
    src_p = jnp.concatenate([src, pad]).reshape(EP // CH, CH)
    dst_p = jnp.concatenate([dst, pad]).reshape(EP // CH, CH)
    x_p = jnp.concatenate([x, jnp.zeros((NP - N, D), x.dtype)])
    z1 = jnp.zeros((NP,), jnp.float32)
    z2 = jnp.zeros((NP, H), jnp.float32)

    deg0, deg1 = _sc_deg(dst_p, z1)
    p0 = deg0.reshape(NP, 1)
    p1 = deg1.reshape(NP, 1)

    u = _tc_a(x_p, W1, p0, p1)
    agg1 = _sc_agg(u, src_p, dst_p, z2)

    W2p = jnp.concatenate([W2, jnp.zeros((H, H - C), jnp.float32)], axis=1)
    b2p = jnp.concatenate([b2, jnp.zeros((H - C,), jnp.float32)]).reshape(1, H)
    v = _tc_b(agg1[0], agg1[1], u, p0, p1, gamma.reshape(1, H),
              beta.reshape(1, H), b1.reshape(1, H), W2p)

    agg2 = _sc_agg(v, src_p, dst_p, z2)
    outp = _tc_c(agg2[0], agg2[1], v, p0, p1, b2p)
    return outp[:N, :C]
